# Initial kernel scaffold; baseline (speedup 1.0000x reference)
import jax
import jax.numpy as jnp
from jax.experimental import pallas as pl


def kernel(pred_offsets, pred_classes, default_boxes, gt_loc, gt_classes):
    raise NotImplementedError("write your pallas kernel here")


# speedup vs baseline: 5.6574x; 5.6574x over previous
import jax
import jax.numpy as jnp
from jax.experimental import pallas as pl
from jax.experimental.pallas import tpu as pltpu

B, N, C, G = 32, 8732, 81, 32
NEG_RATIO = 3
LOC_W = 1.0
CLS_W = 1.0
IOU_T = 0.5
V0, V1 = 0.1, 0.2

LANES = 128
NCH = (N + LANES - 1) // LANES
NPAD = NCH * LANES
NBK = 1024
NBLKS = (N + NBK - 1) // NBK


def _match_kernel(dbT_ref, poT_ref, gt_ref, gtc_ref,
                  pos_ref, gcls_ref, npos_ref, loc_ref,
                  miou_scr, ll_scr):
    f32 = jnp.float32
    i32 = jnp.int32

    gcx = gt_ref[:, 0:1]
    gcy = gt_ref[:, 1:2]
    gw = gt_ref[:, 2:3]
    gh = gt_ref[:, 3:4]
    gx1 = gcx - gw / 2
    gy1 = gcy - gh / 2
    gx2 = gcx + gw / 2
    gy2 = gcy + gh / 2
    area_g = (gx2 - gx1) * (gy2 - gy1)
    gtc = gtc_ref[...]
    iota_g = jax.lax.broadcasted_iota(i32, (G, 1), 0)

    def chunk(c, carry):
        dmin, dlin = carry
        lin = c * LANES + jax.lax.broadcasted_iota(i32, (1, LANES), 1)
        valid = lin < N

        dcx = dbT_ref[0, pl.ds(c, 1), :]
        dcy = dbT_ref[1, pl.ds(c, 1), :]
        dw = dbT_ref[2, pl.ds(c, 1), :]
        dh = dbT_ref[3, pl.ds(c, 1), :]
        dx1 = dcx - dw / 2
        dy1 = dcy - dh / 2
        dx2 = dcx + dw / 2
        dy2 = dcy + dh / 2
        area_d = (dx2 - dx1) * (dy2 - dy1)

        ltx = jnp.maximum(dx1, gx1)
        lty = jnp.maximum(dy1, gy1)
        rbx = jnp.minimum(dx2, gx2)
        rby = jnp.minimum(dy2, gy2)
        iw = jnp.maximum(rbx - ltx, 0.0)
        ih = jnp.maximum(rby - lty, 0.0)
        inter = iw * ih
        iou = inter / (area_d + area_g - inter)
        iou = jnp.where(valid, iou, -1.0)

        mi = jnp.max(iou, axis=0, keepdims=True)
        gidx = jnp.min(jnp.where(iou == mi, iota_g, G), axis=0,
                       keepdims=True)
        onehot = gidx == iota_g
        gcls = jnp.sum(jnp.where(onehot, gtc, 0), axis=0, keepdims=True)
        mcx = jnp.sum(jnp.where(onehot, gcx, 0.0), axis=0, keepdims=True)
        mcy = jnp.sum(jnp.where(onehot, gcy, 0.0), axis=0, keepdims=True)
        mw = jnp.sum(jnp.where(onehot, gw, 0.0), axis=0, keepdims=True)
        mh = jnp.sum(jnp.where(onehot, gh, 0.0), axis=0, keepdims=True)

        dd = jnp.sqrt((dcx - gcx) ** 2 + (dcy - gcy) ** 2
                      + (dw - gw) ** 2 + (dh - gh) ** 2)
        dd = jnp.where(valid, dd, jnp.inf)
        cmin = jnp.min(dd, axis=1, keepdims=True)
        clin = jnp.min(jnp.where(dd == cmin, jnp.broadcast_to(lin, (G, LANES)),
                                 jnp.iinfo(i32).max), axis=1, keepdims=True)
        upd = cmin < dmin
        dlin = jnp.where(upd, clin, dlin)
        dmin = jnp.minimum(cmin, dmin)

        o0 = (mcx - dcx) / (V0 * dw)
        o1 = (mcy - dcy) / (V0 * dh)
        o2 = jnp.log(mw / dw) / V1
        o3 = jnp.log(mh / dh) / V1
        ll = jnp.zeros((1, LANES), f32)
        for k, o in enumerate((o0, o1, o2, o3)):
            d = jnp.abs(poT_ref[k, pl.ds(c, 1), :] - o)
            ll = ll + jnp.where(d < 1.0, 0.5 * d * d, d - 0.5)

        miou_scr[pl.ds(c, 1), :] = mi
        ll_scr[pl.ds(c, 1), :] = ll
        gcls_ref[pl.ds(c, 1), :] = gcls
        return dmin, dlin

    dmin0 = jnp.full((G, 1), jnp.inf, f32)
    dlin0 = jnp.zeros((G, 1), i32)
    _, dlin = jax.lax.fori_loop(0, NCH, chunk, (dmin0, dlin0))

    miou = miou_scr[...]
    mask_iou = miou > IOU_T
    haspos = jnp.sum(mask_iou.astype(f32)) > 0.0
    lin2 = (jax.lax.broadcasted_iota(i32, (NCH, LANES), 0) * LANES
            + jax.lax.broadcasted_iota(i32, (NCH, LANES), 1))
    fb = jnp.any(lin2[None, :, :] == dlin[:, :, None], axis=0)
    posf = jnp.where(haspos, mask_iou.astype(f32), fb.astype(f32))
    pos_ref[...] = posf
    npos_ref[...] = jnp.sum(posf).reshape(1, 1)
    loc_ref[...] = jnp.sum(jnp.where(posf > 0.0, ll_scr[...], 0.0)).reshape(1, 1)


def _cls_kernel(pc_ref, gcls_ref, posf_ref, nl_ref, cep_ref):
    i32 = jnp.int32
    j = pl.program_id(1)
    x = pc_ref[...]
    m = jnp.max(x, axis=1, keepdims=True)
    e = jnp.exp(x - m)
    s = jnp.sum(e, axis=1, keepdims=True)
    lse = m + jnp.log(s)
    nl = lse - x[:, 0:1]
    nl_ref[...] = nl
    gc = gcls_ref[...]
    iota_c = jax.lax.broadcasted_iota(i32, (NBK, C), 1)
    zgc = jnp.sum(jnp.where(iota_c == gc, x, 0.0), axis=1, keepdims=True)
    row = j * NBK + jax.lax.broadcasted_iota(i32, (NBK, 1), 0)
    take = (posf_ref[...] > 0.0) & (row < N)
    cep_ref[...] = jnp.sum(jnp.where(take, lse - zgc, 0.0)).reshape(1, 1)


def _hardneg_kernel(nl_ref, pos_ref, nh_ref, out_ref):
    f32 = jnp.float32
    i32 = jnp.int32
    lin = (jax.lax.broadcasted_iota(i32, (NCH, LANES), 0) * LANES
           + jax.lax.broadcasted_iota(i32, (NCH, LANES), 1))
    neg = (lin < N) & (pos_ref[...] == 0.0)
    nl = nl_ref[...]
    bits = jax.lax.bitcast_convert_type(nl, i32)
    key = jnp.where(bits < 0, bits ^ jnp.int32(0x7FFFFFFF), bits)
    navail = jnp.sum(neg.astype(i32)).reshape(1, 1)
    k = jnp.minimum(nh_ref[...], navail)

    lo0 = jnp.full((1, 1), jnp.iinfo(i32).min, i32)
    hi0 = jnp.full((1, 1), jnp.iinfo(i32).max, i32)

    def step(_, carry):
        lo, hi = carry
        mid = (lo >> 1) + (hi >> 1) + ((lo & 1) | (hi & 1))
        cnt = jnp.sum((neg & (key >= mid)).astype(i32)).reshape(1, 1)
        p = cnt >= k
        lo = jnp.where(p, mid, lo)
        hi = jnp.where(p, hi, mid - 1)
        return lo, hi

    lo, _ = jax.lax.fori_loop(0, 32, step, (lo0, hi0))
    above = neg & (key > lo)
    cgt = jnp.sum(above.astype(i32)).reshape(1, 1)
    sgt = jnp.sum(jnp.where(above, nl, 0.0)).reshape(1, 1)
    vbits = jnp.where(lo < 0, lo ^ jnp.int32(0x7FFFFFFF), lo)
    fv = jax.lax.bitcast_convert_type(vbits, f32)
    res = sgt + (k - cgt).astype(f32) * fv
    out_ref[...] = jnp.where(k > 0, res, jnp.zeros((1, 1), f32))


def kernel(pred_offsets, pred_classes, default_boxes, gt_loc, gt_classes):
    f32 = jnp.float32
    pad = NPAD - N
    dbT = jnp.pad(default_boxes, ((0, 0), (0, pad), (0, 0)),
                  constant_values=0.5).transpose(0, 2, 1).reshape(B, 4, NCH, LANES)
    poT = jnp.pad(pred_offsets, ((0, 0), (0, pad), (0, 0))
                  ).transpose(0, 2, 1).reshape(B, 4, NCH, LANES)
    gtc3 = gt_classes.astype(jnp.int32).reshape(B, G, 1)

    posf, gcls, npos, locs = pl.pallas_call(
        _match_kernel,
        grid=(B,),
        in_specs=[
            pl.BlockSpec((None, 4, NCH, LANES), lambda b: (b, 0, 0, 0)),
            pl.BlockSpec((None, 4, NCH, LANES), lambda b: (b, 0, 0, 0)),
            pl.BlockSpec((None, G, 4), lambda b: (b, 0, 0)),
            pl.BlockSpec((None, G, 1), lambda b: (b, 0, 0)),
        ],
        out_specs=[
            pl.BlockSpec((None, NCH, LANES), lambda b: (b, 0, 0)),
            pl.BlockSpec((None, NCH, LANES), lambda b: (b, 0, 0)),
            pl.BlockSpec((None, 1, 1), lambda b: (b, 0, 0)),
            pl.BlockSpec((None, 1, 1), lambda b: (b, 0, 0)),
        ],
        out_shape=[
            jax.ShapeDtypeStruct((B, NCH, LANES), f32),
            jax.ShapeDtypeStruct((B, NCH, LANES), jnp.int32),
            jax.ShapeDtypeStruct((B, 1, 1), f32),
            jax.ShapeDtypeStruct((B, 1, 1), f32),
        ],
        scratch_shapes=[
            pltpu.VMEM((NCH, LANES), f32),
            pltpu.VMEM((NCH, LANES), f32),
        ],
    )(dbT, poT, gt_loc, gtc3)

    gcls_n = gcls.reshape(B, NPAD)[:, :N].reshape(B, N, 1)
    posf_n = posf.reshape(B, NPAD)[:, :N].reshape(B, N, 1)

    nl, cep = pl.pallas_call(
        _cls_kernel,
        grid=(B, NBLKS),
        in_specs=[
            pl.BlockSpec((None, NBK, C), lambda b, j: (b, j, 0)),
            pl.BlockSpec((None, NBK, 1), lambda b, j: (b, j, 0)),
            pl.BlockSpec((None, NBK, 1), lambda b, j: (b, j, 0)),
        ],
        out_specs=[
            pl.BlockSpec((None, NBK, 1), lambda b, j: (b, j, 0)),
            pl.BlockSpec((None, None, 1, 1), lambda b, j: (b, j, 0, 0)),
        ],
        out_shape=[
            jax.ShapeDtypeStruct((B, N, 1), f32),
            jax.ShapeDtypeStruct((B, NBLKS, 1, 1), f32),
        ],
    )(pred_classes, gcls_n, posf_n)

    nl3 = jnp.pad(nl.reshape(B, N), ((0, 0), (0, pad))).reshape(B, NCH, LANES)
    nhard = (NEG_RATIO * npos).astype(jnp.int32)

    negce = pl.pallas_call(
        _hardneg_kernel,
        grid=(B,),
        in_specs=[
            pl.BlockSpec((None, NCH, LANES), lambda b: (b, 0, 0)),
            pl.BlockSpec((None, NCH, LANES), lambda b: (b, 0, 0)),
            pl.BlockSpec((None, 1, 1), lambda b: (b, 0, 0)),
        ],
        out_specs=pl.BlockSpec((None, 1, 1), lambda b: (b, 0, 0)),
        out_shape=jax.ShapeDtypeStruct((B, 1, 1), f32),
    )(nl3, posf, nhard)

    n_pos = jnp.sum(npos)
    cls_loss = (jnp.sum(cep) + jnp.sum(negce)) / n_pos
    loc_loss = jnp.sum(locs) / n_pos
    total = LOC_W * loc_loss + CLS_W * cls_loss
    return (total, loc_loss, cls_loss)
